# Initial kernel scaffold; baseline (speedup 1.0000x reference)
import jax
import jax.numpy as jnp
from jax.experimental import pallas as pl


def kernel(user_emb, user_log_var, item_emb, item_log_var, user, pos_item, neg_items):
    raise NotImplementedError("write your pallas kernel here")


# speedup vs baseline: 4.6914x; 4.6914x over previous
import functools

import jax
import jax.numpy as jnp
from jax import lax
from jax.experimental import pallas as pl
from jax.experimental.pallas import tpu as pltpu
from jax.experimental.pallas import tpu_sc as plsc

NC = 2
NS = 16
NW = NC * NS

B = 16384
D = 16
NNEG = 200
UPW = B // NW
UCHUNK = 8
PAIRS = UCHUNK * NNEG
NCHUNK = UPW // UCHUNK
GROUPS = PAIRS // 16
EPS = 1e-10


def _body(uemb, ulv_t, iemb, ilv_t, user, pos, negf,
          pos_out, neg_out,
          uidx_v, pidx_v, ue_v, ulv_v, pe_v, plv_v, w_v, cb_v, posout_v,
          nidx_v, ne_v, nlv_v, nout_v, sem):
    wid = lax.axis_index("s") * NC + lax.axis_index("c")
    base = wid * UPW
    iota = lax.iota(jnp.int32, 16)

    pltpu.sync_copy(user.at[pl.ds(base, UPW)], uidx_v)
    pltpu.sync_copy(pos.at[pl.ds(base, UPW)], pidx_v)
    pltpu.async_copy(uemb.at[uidx_v], ue_v, sem).wait()
    pltpu.async_copy(ulv_t.at[uidx_v], ulv_v, sem).wait()
    pltpu.async_copy(iemb.at[pidx_v], pe_v, sem).wait()
    pltpu.async_copy(ilv_t.at[pidx_v], plv_v, sem).wait()

    def wbody(i, carry):
        row = ulv_v[i]
        w_v[i] = 1.0 / (jnp.exp(row) + EPS)
        return carry

    lax.fori_loop(0, UPW, wbody, 0)

    def cbody(g, carry):
        u16 = g * 16 + iota
        acc = jnp.zeros(16, jnp.float32)
        for d in range(D):
            dl = jnp.full((16,), d, jnp.int32)
            l_d = plsc.load_gather(ulv_v, [u16, dl])
            r = EPS / jnp.exp(l_d)
            acc = acc + (l_d + (r - 0.5 * r * r))
        cb_v[pl.ds(g * 16, 16)] = 0.5 * (acc + float(D))
        return carry

    lax.fori_loop(0, UPW // 16, cbody, 0)

    def pbody(g, carry):
        u16 = g * 16 + iota
        acc = jnp.zeros(16, jnp.float32)
        for d in range(D):
            dl = jnp.full((16,), d, jnp.int32)
            w_d = plsc.load_gather(w_v, [u16, dl])
            u_d = plsc.load_gather(ue_v, [u16, dl])
            p_d = plsc.load_gather(pe_v, [u16, dl])
            v_d = plsc.load_gather(plv_v, [u16, dl])
            du = u_d - p_d
            acc = acc + (w_d * (jnp.exp(v_d) + du * du) + v_d)
        cb = plsc.load_gather(cb_v, [u16])
        posout_v[pl.ds(g * 16, 16)] = 0.5 * acc - cb
        return carry

    lax.fori_loop(0, UPW // 16, pbody, 0)
    pltpu.sync_copy(posout_v, pos_out.at[pl.ds(base, UPW)])

    def nchunk(c, carry):
        pair_base = base * NNEG + c * PAIRS
        pltpu.sync_copy(negf.at[pl.ds(pair_base, PAIRS)], nidx_v)
        pltpu.async_copy(iemb.at[nidx_v], ne_v, sem).wait()
        pltpu.async_copy(ilv_t.at[nidx_v], nlv_v, sem).wait()

        def gbody(g, gcarry):
            p16 = g * 16 + iota
            u16 = (c * PAIRS + p16) // NNEG
            acc = jnp.zeros(16, jnp.float32)
            for d in range(D):
                dl = jnp.full((16,), d, jnp.int32)
                x_d = plsc.load_gather(ne_v, [p16, dl])
                v_d = plsc.load_gather(nlv_v, [p16, dl])
                w_d = plsc.load_gather(w_v, [u16, dl])
                u_d = plsc.load_gather(ue_v, [u16, dl])
                du = u_d - x_d
                acc = acc + (w_d * (jnp.exp(v_d) + du * du) + v_d)
            cb = plsc.load_gather(cb_v, [u16])
            nout_v[pl.ds(g * 16, 16)] = 0.5 * acc - cb
            return gcarry

        lax.fori_loop(0, GROUPS, gbody, 0)
        pltpu.sync_copy(nout_v, neg_out.at[pl.ds(pair_base, PAIRS)])
        return carry

    lax.fori_loop(0, NCHUNK, nchunk, 0)


@jax.jit
def _sc_forward(uemb, ulv_t, iemb, ilv_t, user, pos, negf):
    return pl.kernel(
        _body,
        out_type=[
            jax.ShapeDtypeStruct((B,), jnp.float32),
            jax.ShapeDtypeStruct((B * NNEG,), jnp.float32),
        ],
        mesh=plsc.VectorSubcoreMesh(core_axis_name="c", subcore_axis_name="s"),
        compiler_params=pltpu.CompilerParams(
            needs_layout_passes=False, use_tc_tiling_on_sc=False),
        scratch_types=[
            pltpu.VMEM((UPW,), jnp.int32),
            pltpu.VMEM((UPW,), jnp.int32),
            pltpu.VMEM((UPW, D), jnp.float32),
            pltpu.VMEM((UPW, D), jnp.float32),
            pltpu.VMEM((UPW, D), jnp.float32),
            pltpu.VMEM((UPW, D), jnp.float32),
            pltpu.VMEM((UPW, D), jnp.float32),
            pltpu.VMEM((UPW,), jnp.float32),
            pltpu.VMEM((UPW,), jnp.float32),
            pltpu.VMEM((PAIRS,), jnp.int32),
            pltpu.VMEM((PAIRS, D), jnp.float32),
            pltpu.VMEM((PAIRS, D), jnp.float32),
            pltpu.VMEM((PAIRS,), jnp.float32),
            pltpu.SemaphoreType.DMA,
        ],
    )(uemb, ulv_t, iemb, ilv_t, user, pos, negf)


def kernel(user_emb, user_log_var, item_emb, item_log_var, user, pos_item, neg_items):
    negf = neg_items.reshape(-1).astype(jnp.int32)
    pos_s, neg_s = _sc_forward(
        user_emb, user_log_var, item_emb, item_log_var,
        user.astype(jnp.int32), pos_item.astype(jnp.int32), negf)
    return pos_s.reshape(B, 1), neg_s.reshape(B, NNEG)


# speedup vs baseline: 5.9692x; 1.2724x over previous
import jax
import jax.numpy as jnp
from jax import lax
from jax.experimental import pallas as pl
from jax.experimental.pallas import tpu as pltpu
from jax.experimental.pallas import tpu_sc as plsc

NC = 2
NS = 16
NW = NC * NS

B = 16384
D = 16
NNEG = 200
UPW = B // NW
UBLK = UPW // 16
JC = 50
EPS = 1e-10


def _main_body(uemb, ulv_t, emb_rm, lv_rm, user, pos, negT,
               pos_out, neg_out,
               uidx_v, pidx_v, ue_v, ulv_v, ueT_v, wT_v, cb0_v, cb2_v,
               pe_v, plv_v, posout_v, nidxb_v, nidx1_v, ne_v, nlv_v, nout_v,
               semu, semg0, semg1, semx0, semx1):
    wid = lax.axis_index("s") * NC + lax.axis_index("c")
    base = wid * UPW
    iota = lax.iota(jnp.int32, 16)
    semg = [semg0, semg1]
    semx = [semx0, semx1]

    pltpu.sync_copy(user.at[pl.ds(base, UPW)], uidx_v)
    pltpu.sync_copy(pos.at[pl.ds(base, UPW)], pidx_v)

    pltpu.async_copy(uemb.at[uidx_v], ue_v, semu)
    pltpu.async_copy(ulv_t.at[uidx_v], ulv_v, semu)
    pltpu.async_copy(emb_rm.at[pidx_v], pe_v, semu)
    pltpu.async_copy(lv_rm.at[pidx_v], plv_v, semu)
    for buf in (ue_v, ulv_v, pe_v, plv_v):
        pltpu.make_async_copy(emb_rm.at[pl.ds(0, UPW), :], buf, semu).wait()

    def cprep(g, carry):
        sl = pl.ds(g * 16, 16)
        u16 = g * 16 + iota
        acc_c = jnp.zeros(16, jnp.float32)
        acc_w = jnp.zeros(16, jnp.float32)
        for d in range(D):
            dl = jnp.full((16,), d, jnp.int32)
            l_d = plsc.load_gather(ulv_v, [u16, dl])
            u_d = plsc.load_gather(ue_v, [u16, dl])
            e_d = jnp.exp(l_d)
            w_d = 1.0 / (e_d + EPS)
            wT_v[d, sl] = w_d
            ueT_v[d, sl] = u_d
            r = EPS / e_d
            acc_c = acc_c + (l_d + (r - 0.5 * r * r))
            acc_w = acc_w + w_d * u_d * u_d
        c0 = 0.5 * (acc_c + float(D))
        cb0_v[sl] = c0
        cb2_v[sl] = c0 - 0.5 * acc_w
        return carry

    lax.fori_loop(0, UBLK, cprep, 0)

    def pbody(g, carry):
        sl = pl.ds(g * 16, 16)
        u16 = g * 16 + iota
        acc = jnp.zeros(16, jnp.float32)
        for d in range(D):
            dl = jnp.full((16,), d, jnp.int32)
            p_d = plsc.load_gather(pe_v, [u16, dl])
            v_d = plsc.load_gather(plv_v, [u16, dl])
            w_d = wT_v[d, sl]
            u_d = ueT_v[d, sl]
            du = u_d - p_d
            acc = acc + (w_d * (jnp.exp(v_d) + du * du) + v_d)
        posout_v[sl] = 0.5 * acc - cb0_v[sl]
        return carry

    lax.fori_loop(0, UBLK, pbody, 0)
    pltpu.sync_copy(posout_v, pos_out.at[pl.ds(base, UPW)])

    out_stride = iota * NNEG

    def flatten_and_fire(ubpar, qq, par):
        def flat(i, fcarry):
            nidx1_v[par, pl.ds(i * 16, 16)] = nidxb_v[ubpar, qq * JC + i]
            return fcarry

        lax.fori_loop(0, JC, flat, 0)
        pltpu.async_copy(emb_rm.at[nidx1_v.at[par]], ne_v.at[par], semg[par])
        pltpu.async_copy(lv_rm.at[nidx1_v.at[par]], nlv_v.at[par], semg[par])

    def drain_gathers(par):
        pltpu.make_async_copy(emb_rm.at[pl.ds(0, JC * 16), :], ne_v.at[par], semg[par]).wait()
        pltpu.make_async_copy(lv_rm.at[pl.ds(0, JC * 16), :], nlv_v.at[par], semg[par]).wait()

    def quarter(ub, ubpar, qq, par):
        ucol = base + ub * 16
        sl = pl.ds(ub * 16, 16)

        drain_gathers(par)

        if qq == 0:
            @pl.when(ub < UBLK - 1)
            def _stage():
                pltpu.async_copy(negT.at[:, pl.ds(ucol + 16, 16)],
                                 nidxb_v.at[1 - ubpar], semx[1 - ubpar])

        if qq < 3:
            flatten_and_fire(ubpar, qq + 1, 1 - par)
        else:
            @pl.when(ub < UBLK - 1)
            def _next_block():
                pltpu.make_async_copy(negT.at[:, pl.ds(0, 16)],
                                      nidxb_v.at[1 - ubpar], semx[1 - ubpar]).wait()
                flatten_and_fire(1 - ubpar, 0, 1 - par)

        ab = []
        for d in range(D):
            a_d = wT_v[d, sl]
            b_d = (-2.0) * a_d * ueT_v[d, sl]
            ab.append((a_d, b_d))
        cb2 = cb2_v[sl]
        j0 = qq * JC

        def jbody(jj, jcarry):
            p16 = jj * 16 + iota
            acc = jnp.zeros(16, jnp.float32)
            for d in range(D):
                dl = jnp.full((16,), d, jnp.int32)
                x = plsc.load_gather(ne_v.at[par], [p16, dl])
                v = plsc.load_gather(nlv_v.at[par], [p16, dl])
                a_d, b_d = ab[d]
                acc = acc + (a_d * (jnp.exp(v) + x * x) + b_d * x + v)
            score = 0.5 * acc - cb2
            plsc.store_scatter(nout_v, [out_stride + (j0 + jj)], score)
            return jcarry

        lax.fori_loop(0, JC, jbody, 0)

        if qq == 3:
            pltpu.sync_copy(nout_v, neg_out.at[pl.ds(ucol * NNEG, 16 * NNEG)])

    pltpu.async_copy(negT.at[:, pl.ds(base, 16)], nidxb_v.at[0], semx[0])
    pltpu.make_async_copy(negT.at[:, pl.ds(0, 16)], nidxb_v.at[0], semx[0]).wait()
    flatten_and_fire(0, 0, 0)

    def octet(h, carry):
        for sub in range(2):
            ub = h * 2 + sub
            ubpar = sub
            for qq in range(4):
                par = (qq + sub * 4) % 2
                quarter(ub, ubpar, qq, par)
        return carry

    lax.fori_loop(0, UBLK // 2, octet, 0)


_SC_PARAMS = pltpu.CompilerParams(
    needs_layout_passes=False, use_tc_tiling_on_sc=False)
_MESH = dict(core_axis_name="c", subcore_axis_name="s")


@jax.jit
def _forward(uemb, ulv_t, iemb, ilv_t, user, pos, negT):
    return pl.kernel(
        _main_body,
        out_type=[
            jax.ShapeDtypeStruct((B,), jnp.float32),
            jax.ShapeDtypeStruct((B * NNEG,), jnp.float32),
        ],
        mesh=plsc.VectorSubcoreMesh(**_MESH),
        compiler_params=_SC_PARAMS,
        scratch_types=[
            pltpu.VMEM((UPW,), jnp.int32),
            pltpu.VMEM((UPW,), jnp.int32),
            pltpu.VMEM((UPW, D), jnp.float32),
            pltpu.VMEM((UPW, D), jnp.float32),
            pltpu.VMEM((D, UPW), jnp.float32),
            pltpu.VMEM((D, UPW), jnp.float32),
            pltpu.VMEM((UPW,), jnp.float32),
            pltpu.VMEM((UPW,), jnp.float32),
            pltpu.VMEM((UPW, D), jnp.float32),
            pltpu.VMEM((UPW, D), jnp.float32),
            pltpu.VMEM((UPW,), jnp.float32),
            pltpu.VMEM((2, NNEG, 16), jnp.int32),
            pltpu.VMEM((2, JC * 16), jnp.int32),
            pltpu.VMEM((2, JC * 16, D), jnp.float32),
            pltpu.VMEM((2, JC * 16, D), jnp.float32),
            pltpu.VMEM((16 * NNEG,), jnp.float32),
        ] + [pltpu.SemaphoreType.DMA] * 5,
    )(uemb, ulv_t, iemb, ilv_t, user, pos, negT)


def kernel(user_emb, user_log_var, item_emb, item_log_var, user, pos_item, neg_items):
    pos_s, neg_s = _forward(
        user_emb, user_log_var, item_emb, item_log_var,
        user.astype(jnp.int32), pos_item.astype(jnp.int32),
        neg_items.T.astype(jnp.int32))
    return pos_s.reshape(B, 1), neg_s.reshape(B, NNEG)
